# baseline (speedup 1.0000x reference)
import functools

import jax
import jax.numpy as jnp
from jax import lax
from jax.experimental import pallas as pl
from jax.experimental.pallas import tpu as pltpu
from jax.experimental.pallas import tpu_sc as plsc

_B = 16384
_EMB = 128
_REL = 1000
_NC = 2
_NS = 16
_NW = _NC * _NS
_BPW = _B // _NW
_C = 64
_NCHUNK = _BPW // _C
_NBUF = 3
_L = 16


def _build():
    mesh = plsc.VectorSubcoreMesh(core_axis_name="c", subcore_axis_name="s")

    @functools.partial(
        pl.kernel,
        mesh=mesh,
        compiler_params=pltpu.CompilerParams(needs_layout_passes=False),
        out_type=jax.ShapeDtypeStruct((_B,), jnp.float32),
        scratch_types=[
            pltpu.VMEM((2 * _BPW,), jnp.int32),
            pltpu.VMEM((_BPW,), jnp.int32),
            *[pltpu.VMEM((2 * _C, _EMB), jnp.float32) for _ in range(_NBUF)],
            pltpu.VMEM((_REL * _EMB // 2,), jnp.int32),
            pltpu.VMEM((_L * _L,), jnp.float32),
            pltpu.VMEM((_BPW,), jnp.float32),
            pltpu.VMEM_SHARED((_REL * _EMB // 2,), jnp.int32),
            pltpu.SemaphoreType.DMA,
            pltpu.SemaphoreType.DMA,
            pltpu.SemaphoreType.DMA,
        ],
    )
    def transe(ht_hbm, r_hbm, ent_hbm, relq_hbm, out_hbm,
               htidx, ridx, buf0, buf1, buf2,
               rel_local, accbuf, scores, rel_sh, sem0, sem1, sem2):
        sid = lax.axis_index("s")
        wid = sid * _NC + lax.axis_index("c")
        base = wid * _BPW
        pltpu.sync_copy(ht_hbm.at[pl.ds(2 * base, 2 * _BPW)], htidx)
        pltpu.sync_copy(r_hbm.at[pl.ds(base, _BPW)], ridx)

        bufs = ((buf0, sem0), (buf1, sem1), (buf2, sem2))
        lanes = lax.iota(jnp.int32, _L)

        def issue(ci):
            buf, sem = bufs[ci % _NBUF]
            return pltpu.async_copy(
                ent_hbm.at[htidx.at[pl.ds(ci * 2 * _C, 2 * _C)]], buf, sem)

        descs = [issue(0), issue(1), issue(2)]

        @pl.when(sid == 0)
        def _():
            pltpu.sync_copy(relq_hbm, rel_sh)

        plsc.subcore_barrier()
        pltpu.sync_copy(rel_sh, rel_local)

        def compute(ci):
            buf, _ = bufs[ci % _NBUF]
            off = ci * _C

            def group(g, carry):
                rbase = g * _L
                rvec = ridx[pl.ds(off + rbase, _L)]
                for i in range(_L):
                    rword = rvec[i] * (_EMB // 2)
                    acc = jnp.zeros((_L,), jnp.float32)
                    for m in range(_EMB // (2 * _L)):
                        rpacked = rel_local[pl.ds(rword + m * _L, _L)]
                        rpair = plsc.bitcast(rpacked, jnp.bfloat16)
                        ra, rb = plsc.unpack(
                            rpair, format=plsc.PackFormat.INTERLEAVED)
                        for half, rv in ((0, ra), (1, rb)):
                            j = 2 * m + half
                            hv = buf[rbase + i, pl.ds(j * _L, _L)]
                            tv = buf[_C + rbase + i, pl.ds(j * _L, _L)]
                            d = (hv - tv) + rv
                            acc = acc + d * d
                    accbuf[pl.ds(i * _L, _L)] = acc
                sv = jnp.zeros((_L,), jnp.float32)
                for k in range(_L):
                    sv = sv + plsc.load_gather(accbuf, [lanes * _L + k])
                scores[pl.ds(off + g * _L, _L)] = sv
                return carry

            lax.fori_loop(0, _C // _L, group, 0)

        for ci in range(_NCHUNK):
            descs[0].wait()
            descs = descs[1:]
            compute(ci)
            if ci + _NBUF < _NCHUNK:
                descs.append(issue(ci + _NBUF))

        pltpu.sync_copy(scores, out_hbm.at[pl.ds(base, _BPW)])

    return transe


_TRANSE = _build()


def kernel(h, r, t, ent_emb, rel_emb):
    hh = h.astype(jnp.int32).reshape(_NW, _NCHUNK, _C)
    tt = t.astype(jnp.int32).reshape(_NW, _NCHUNK, _C)
    ht = jnp.stack([hh, tt], axis=2).reshape(2 * _B)
    relq = (rel_emb.reshape(_REL, _EMB // 32, 2, _L)
            .transpose(0, 1, 3, 2)
            .reshape(_REL * _EMB // 2, 2)
            .astype(jnp.bfloat16))
    relq_i32 = lax.bitcast_convert_type(relq, jnp.int32)
    return _TRANSE(ht, r.astype(jnp.int32), ent_emb, relq_i32)


# speedup vs baseline: 1.1426x; 1.1426x over previous
import functools

import jax
import jax.numpy as jnp
from jax import lax
from jax.experimental import pallas as pl
from jax.experimental.pallas import tpu as pltpu
from jax.experimental.pallas import tpu_sc as plsc

_B = 16384
_EMB = 128
_REL = 1000
_NC = 2
_NS = 16
_NW = _NC * _NS
_BPW = _B // _NW
_C = 64
_NCHUNK = _BPW // _C
_NBUF = 3
_L = 16


def _build():
    mesh = plsc.VectorSubcoreMesh(core_axis_name="c", subcore_axis_name="s")

    @functools.partial(
        pl.kernel,
        mesh=mesh,
        compiler_params=pltpu.CompilerParams(needs_layout_passes=False),
        out_type=jax.ShapeDtypeStruct((_B,), jnp.float32),
        scratch_types=[
            pltpu.VMEM((2 * _BPW,), jnp.int32),
            pltpu.VMEM((_BPW,), jnp.int32),
            *[pltpu.VMEM((2 * _C, _EMB), jnp.float32) for _ in range(_NBUF)],
            pltpu.VMEM((_REL * _EMB // 2,), jnp.int32),
            pltpu.VMEM((_L * _L,), jnp.float32),
            pltpu.VMEM((_BPW,), jnp.float32),
            pltpu.VMEM_SHARED((_REL * _EMB // 2,), jnp.int32),
            pltpu.SemaphoreType.DMA,
            pltpu.SemaphoreType.DMA,
            pltpu.SemaphoreType.DMA,
        ],
    )
    def transe(ht_hbm, r_hbm, ent_hbm, relq_hbm, out_hbm,
               htidx, ridx, buf0, buf1, buf2,
               rel_local, accbuf, scores, rel_sh, sem0, sem1, sem2):
        sid = lax.axis_index("s")
        wid = sid * _NC + lax.axis_index("c")
        base = wid * _BPW
        pltpu.sync_copy(ht_hbm.at[pl.ds(2 * base, 2 * _BPW)], htidx)
        pltpu.sync_copy(r_hbm.at[pl.ds(base, _BPW)], ridx)

        bufs = ((buf0, sem0), (buf1, sem1), (buf2, sem2))
        lanes = lax.iota(jnp.int32, _L)

        def issue(ci):
            buf, sem = bufs[ci % _NBUF]
            return pltpu.async_copy(
                ent_hbm.at[htidx.at[pl.ds(ci * 2 * _C, 2 * _C)]], buf, sem)

        descs = [issue(0), issue(1), issue(2)]

        @pl.when(sid == 0)
        def _():
            pltpu.sync_copy(relq_hbm, rel_sh)

        plsc.subcore_barrier()
        pltpu.sync_copy(rel_sh, rel_local)

        def compute(ci):
            buf, _ = bufs[ci % _NBUF]
            off = ci * _C

            def group(g, carry):
                rbase = g * _L
                rvec = ridx[pl.ds(off + rbase, _L)]
                sv = jnp.zeros((_L,), jnp.float32)
                for i in range(_L):
                    rword = rvec[i] * (_EMB // 2)
                    acc = jnp.zeros((_L,), jnp.float32)
                    for m in range(_EMB // (2 * _L)):
                        rpacked = rel_local[pl.ds(rword + m * _L, _L)]
                        rpair = plsc.bitcast(rpacked, jnp.bfloat16)
                        ra, rb = plsc.unpack(
                            rpair, format=plsc.PackFormat.INTERLEAVED)
                        for half, rv in ((0, ra), (1, rb)):
                            j = 2 * m + half
                            hv = buf[rbase + i, pl.ds(j * _L, _L)]
                            tv = buf[_C + rbase + i, pl.ds(j * _L, _L)]
                            d = (hv - tv) + rv
                            acc = acc + d * d
                    sv = jnp.where(lanes == i, jnp.sum(acc), sv)
                scores[pl.ds(off + rbase, _L)] = sv
                return carry

            lax.fori_loop(0, _C // _L, group, 0)

        for ci in range(_NCHUNK):
            descs[0].wait()
            descs = descs[1:]
            compute(ci)
            if ci + _NBUF < _NCHUNK:
                descs.append(issue(ci + _NBUF))

        pltpu.sync_copy(scores, out_hbm.at[pl.ds(base, _BPW)])

    return transe


_TRANSE = _build()


def kernel(h, r, t, ent_emb, rel_emb):
    hh = h.astype(jnp.int32).reshape(_NW, _NCHUNK, _C)
    tt = t.astype(jnp.int32).reshape(_NW, _NCHUNK, _C)
    ht = jnp.stack([hh, tt], axis=2).reshape(2 * _B)
    relq = (rel_emb.reshape(_REL, _EMB // 32, 2, _L)
            .transpose(0, 1, 3, 2)
            .reshape(_REL * _EMB // 2, 2)
            .astype(jnp.bfloat16))
    relq_i32 = lax.bitcast_convert_type(relq, jnp.int32)
    return _TRANSE(ht, r.astype(jnp.int32), ent_emb, relq_i32)
